# baseline (speedup 1.0000x reference)
import functools

import jax
import jax.numpy as jnp
from jax import lax
from jax.experimental import pallas as pl
from jax.experimental.pallas import tpu as pltpu
from jax.experimental.pallas import tpu_sc as plsc

NC = 2
NS = 16
NW = NC * NS
LANES = 16

_MESH = dict(core_axis_name="c", subcore_axis_name="s",
             num_cores=NC, num_subcores=NS)


_BLK = 512


def _make_tc_format(V, D):
    nblk = V // _BLK
    vrem = V - nblk * _BLK

    def body(tbl_ref, tail_ref, out_ref):
        pid = pl.program_id(0)

        @pl.when(pid < nblk)
        def _():
            x = tbl_ref[...]
            out_ref[...] = x.T.reshape(_BLK // 2, 2 * D)

        if vrem:
            @pl.when(pid == nblk)
            def _():
                out_ref[pl.ds(0, vrem // 2), :] = tail_ref[...]

    grid = nblk + (1 if vrem else 0)
    return pl.pallas_call(
        body,
        grid=(grid,),
        in_specs=[
            pl.BlockSpec((D, _BLK), lambda c: (0, jnp.minimum(c, nblk - 1))),
            pl.BlockSpec((vrem // 2, 2 * D) if vrem else None,
                         (lambda c: (0, 0)) if vrem else None),
        ],
        out_specs=pl.BlockSpec((_BLK // 2, 2 * D), lambda c: (c, 0)),
        out_shape=jax.ShapeDtypeStruct((V // 2, 2 * D), jnp.float32),
    )


def _make_sc_pool(B, L, D):
    rows_w = B // NW
    CA = 104
    CB = L - CA
    nvec = D // LANES
    scale = 1.0 / L
    nidx = rows_w * L

    @functools.partial(
        pl.kernel,
        out_type=jax.ShapeDtypeStruct((B, D), jnp.float32),
        mesh=plsc.VectorSubcoreMesh(**_MESH),
        compiler_params=pltpu.CompilerParams(use_tc_tiling_on_sc=False),
        scratch_types=[
            pltpu.VMEM((nidx,), jnp.int32),
            pltpu.VMEM((2, L, D), jnp.float32),
            pltpu.VMEM((rows_w, D), jnp.float32),
            pltpu.SemaphoreType.DMA,
            pltpu.SemaphoreType.DMA,
        ],
    )
    def sc_pool(lin_hbm, idx_hbm, out_hbm, idx_v, buf, pooled_v, sem0, sem1):
        wid = lax.axis_index("s") * NC + lax.axis_index("c")
        pltpu.sync_copy(idx_hbm.at[pl.ds(wid * nidx, nidx)], idx_v)
        sems = (sem0, sem1)

        def row_copies(r, b):
            o = r * L
            ca = pltpu.make_async_copy(
                lin_hbm.at[idx_v.at[pl.ds(o, CA)]],
                buf.at[b, pl.ds(0, CA)], sems[b])
            cb = pltpu.make_async_copy(
                lin_hbm.at[idx_v.at[pl.ds(o + CA, CB)]],
                buf.at[b, pl.ds(CA, CB)], sems[b])
            return ca, cb

        def issue(r, b):
            ca, cb = row_copies(r, b)
            ca.start()
            cb.start()

        def wait_row(r, b):
            ca, cb = row_copies(r, b)
            ca.wait()
            cb.wait()

        def acc_row(r, b):
            def jbody(j, accs):
                return tuple(
                    accs[k] + buf[b, j, pl.ds(k * LANES, LANES)]
                    for k in range(nvec))
            z = jnp.zeros((LANES,), jnp.float32)
            accs = lax.fori_loop(0, L, jbody, (z,) * nvec, unroll=8)
            for k in range(nvec):
                pooled_v[r, pl.ds(k * LANES, LANES)] = accs[k] * scale

        issue(0, 0)
        issue(1, 1)

        def obody(rr, carry):
            for b in range(2):
                r = 2 * rr + b
                wait_row(r, b)

                @pl.when(r + 2 < rows_w)
                def _():
                    issue(r + 2, b)

                acc_row(r, b)
            return carry

        lax.fori_loop(0, rows_w // 2, obody, 0)
        pltpu.sync_copy(pooled_v, out_hbm.at[pl.ds(wid * rows_w, rows_w)])

    return sc_pool


def _dense_body(pooled_ref, w1_ref, b1_ref, w2_ref, b2_ref, out_ref):
    p = pooled_ref[...]
    h = jnp.dot(p, w1_ref[...], preferred_element_type=jnp.float32)
    h = jnp.maximum(h + b1_ref[...], 0.0)
    logits = jnp.dot(h, w2_ref[...], preferred_element_type=jnp.float32)
    logits = logits + b2_ref[...]
    m = jnp.max(logits, axis=-1, keepdims=True)
    e = jnp.exp(logits - m)
    out_ref[...] = e / jnp.sum(e, axis=-1, keepdims=True)


def kernel(inputs, emb_table, W1, b1, W2, b2):
    B, L = inputs.shape
    V, D = emb_table.shape
    H = W1.shape[1]
    C = W2.shape[1]
    CP = 128

    idx_flat = inputs.astype(jnp.int32).reshape(-1)
    pair = jnp.concatenate([emb_table[0::2], emb_table[1::2]], axis=1)
    lin = pair.reshape(V, D)
    pooled = _make_sc_pool(B, L, D)(lin, idx_flat)

    w2p = jnp.zeros((H, CP), jnp.float32).at[:, :C].set(W2)
    b2p = jnp.full((1, CP), -1e30, jnp.float32).at[0, :C].set(b2)
    b1r = b1.reshape(1, H)

    out = pl.pallas_call(
        _dense_body,
        out_shape=jax.ShapeDtypeStruct((B, CP), jnp.float32),
    )(pooled, W1, b1r, w2p, b2p)
    return out[:, :C]


# speedup vs baseline: 6.7338x; 6.7338x over previous
import functools

import jax
import jax.numpy as jnp
from jax import lax
from jax.experimental import pallas as pl
from jax.experimental.pallas import tpu as pltpu
from jax.experimental.pallas import tpu_sc as plsc

NC = 2
NS = 16
NW = NC * NS
LANES = 16

_MESH = dict(core_axis_name="c", subcore_axis_name="s",
             num_cores=NC, num_subcores=NS)


_BLK = 512


def _make_tc_format(V, D):
    nblk = V // _BLK
    vrem = V - nblk * _BLK

    def body(tbl_ref, tail_ref, out_ref):
        pid = pl.program_id(0)

        @pl.when(pid < nblk)
        def _():
            t = tbl_ref[...].T
            out_ref[:, pl.ds(0, D)] = t[: _BLK // 2]
            out_ref[:, pl.ds(D, D)] = t[_BLK // 2:]

        if vrem:
            @pl.when(pid == nblk)
            def _():
                out_ref[pl.ds(0, vrem // 2), :] = tail_ref[...]

    grid = nblk + (1 if vrem else 0)
    return pl.pallas_call(
        body,
        grid=(grid,),
        in_specs=[
            pl.BlockSpec((D, _BLK), lambda c: (0, jnp.minimum(c, nblk - 1))),
            pl.BlockSpec((vrem // 2, 2 * D) if vrem else None,
                         (lambda c: (0, 0)) if vrem else None),
        ],
        out_specs=pl.BlockSpec((_BLK // 2, 2 * D), lambda c: (c, 0)),
        out_shape=jax.ShapeDtypeStruct((V // 2, 2 * D), jnp.float32),
    )


def _make_sc_pool(B, L, D, Vt):
    rows_w = B // NW
    CA = 104
    CB = L - CA
    nvec = D // LANES
    scale = 1.0 / L
    nidx = rows_w * L
    half = _BLK // 2
    hshift = half.bit_length() - 1

    @functools.partial(
        pl.kernel,
        out_type=jax.ShapeDtypeStruct((B, D), jnp.float32),
        mesh=plsc.VectorSubcoreMesh(**_MESH),
        compiler_params=pltpu.CompilerParams(use_tc_tiling_on_sc=False),
        scratch_types=[
            pltpu.VMEM((nidx,), jnp.int32),
            pltpu.VMEM((2, L, D), jnp.float32),
            pltpu.VMEM((rows_w, D), jnp.float32),
            pltpu.SemaphoreType.DMA,
            pltpu.SemaphoreType.DMA,
        ],
    )
    def sc_pool(lin_hbm, idx_hbm, out_hbm, idx_v, buf, pooled_v, sem0, sem1):
        wid = lax.axis_index("s") * NC + lax.axis_index("c")
        pltpu.sync_copy(idx_hbm.at[pl.ds(wid * nidx, nidx)], idx_v)
        sems = (sem0, sem1)

        def pbody(i, carry):
            x = idx_v[pl.ds(i * LANES, LANES)]
            p = ((x & (-_BLK)) | ((x & (half - 1)) << 1)
                 | (lax.shift_right_logical(x, hshift) & 1))
            idx_v[pl.ds(i * LANES, LANES)] = jnp.where(x < Vt, p, x)
            return carry
        lax.fori_loop(0, nidx // LANES, pbody, 0, unroll=8)

        def row_copies(r, b):
            o = r * L
            ca = pltpu.make_async_copy(
                lin_hbm.at[idx_v.at[pl.ds(o, CA)]],
                buf.at[b, pl.ds(0, CA)], sems[b])
            cb = pltpu.make_async_copy(
                lin_hbm.at[idx_v.at[pl.ds(o + CA, CB)]],
                buf.at[b, pl.ds(CA, CB)], sems[b])
            return ca, cb

        def issue(r, b):
            ca, cb = row_copies(r, b)
            ca.start()
            cb.start()

        def wait_row(r, b):
            ca, cb = row_copies(r, b)
            ca.wait()
            cb.wait()

        def acc_row(r, b):
            def jbody(j, accs):
                return tuple(
                    accs[k] + buf[b, j, pl.ds(k * LANES, LANES)]
                    for k in range(nvec))
            z = jnp.zeros((LANES,), jnp.float32)
            accs = lax.fori_loop(0, L, jbody, (z,) * nvec, unroll=8)
            for k in range(nvec):
                pooled_v[r, pl.ds(k * LANES, LANES)] = accs[k] * scale

        issue(0, 0)
        issue(1, 1)

        def obody(rr, carry):
            for b in range(2):
                r = 2 * rr + b
                wait_row(r, b)

                @pl.when(r + 2 < rows_w)
                def _():
                    issue(r + 2, b)

                acc_row(r, b)
            return carry

        lax.fori_loop(0, rows_w // 2, obody, 0)
        pltpu.sync_copy(pooled_v, out_hbm.at[pl.ds(wid * rows_w, rows_w)])

    return sc_pool


def _dense_body(pooled_ref, w1_ref, b1_ref, w2_ref, b2_ref, out_ref):
    p = pooled_ref[...]
    h = jnp.dot(p, w1_ref[...], preferred_element_type=jnp.float32)
    h = jnp.maximum(h + b1_ref[...], 0.0)
    logits = jnp.dot(h, w2_ref[...], preferred_element_type=jnp.float32)
    logits = logits + b2_ref[...]
    m = jnp.max(logits, axis=-1, keepdims=True)
    e = jnp.exp(logits - m)
    out_ref[...] = e / jnp.sum(e, axis=-1, keepdims=True)


def kernel(inputs, emb_table, W1, b1, W2, b2):
    B, L = inputs.shape
    V, D = emb_table.shape
    H = W1.shape[1]
    C = W2.shape[1]
    CP = 128

    idx_flat = inputs.astype(jnp.int32).reshape(-1)
    vrem = V - (V // _BLK) * _BLK
    tail = emb_table[V - vrem:].reshape(vrem // 2, 2 * D)
    pair = _make_tc_format(V, D)(emb_table.T, tail)
    lin = pair.reshape(V, D)
    pooled = _make_sc_pool(B, L, D, (V // _BLK) * _BLK)(lin, idx_flat)

    w2p = jnp.zeros((H, CP), jnp.float32).at[:, :C].set(W2)
    b2p = jnp.full((1, CP), -1e30, jnp.float32).at[0, :C].set(b2)
    b1r = b1.reshape(1, H)

    out = pl.pallas_call(
        _dense_body,
        out_shape=jax.ShapeDtypeStruct((B, CP), jnp.float32),
    )(pooled, W1, b1r, w2p, b2p)
    return out[:, :C]


# speedup vs baseline: 15.6403x; 2.3227x over previous
import functools

import jax
import jax.numpy as jnp
from jax import lax
from jax.experimental import pallas as pl
from jax.experimental.pallas import tpu as pltpu
from jax.experimental.pallas import tpu_sc as plsc

NC = 2
NS = 16
NW = NC * NS
LANES = 16

_MESH = dict(core_axis_name="c", subcore_axis_name="s",
             num_cores=NC, num_subcores=NS)


_BLK = 2048


def _make_tc_format(V, D):
    nblk = V // _BLK
    vrem = V - nblk * _BLK

    def body(tbl_ref, tail_ref, out_ref):
        pid = pl.program_id(0)

        @pl.when(pid < nblk)
        def _():
            t = tbl_ref[...].T
            out_ref[:, pl.ds(0, D)] = t[: _BLK // 2]
            out_ref[:, pl.ds(D, D)] = t[_BLK // 2:]

        if vrem:
            @pl.when(pid == nblk)
            def _():
                out_ref[pl.ds(0, vrem // 2), :] = tail_ref[...]

    grid = nblk + (1 if vrem else 0)
    return pl.pallas_call(
        body,
        grid=(grid,),
        in_specs=[
            pl.BlockSpec((D, _BLK), lambda c: (0, jnp.minimum(c, nblk - 1))),
            pl.BlockSpec((vrem // 2, 2 * D) if vrem else None,
                         (lambda c: (0, 0)) if vrem else None),
        ],
        out_specs=pl.BlockSpec((_BLK // 2, 2 * D), lambda c: (c, 0)),
        out_shape=jax.ShapeDtypeStruct((V // 2, 2 * D), jnp.float32),
    )


def _make_sc_pool(B, L, D, Vt):
    rows_w = B // NW
    CA = 104
    CB = L - CA
    nvec = D // LANES
    scale = 1.0 / L
    nidx = rows_w * L
    half = _BLK // 2
    hshift = half.bit_length() - 1

    @functools.partial(
        pl.kernel,
        out_type=jax.ShapeDtypeStruct((B, D), jnp.float32),
        mesh=plsc.VectorSubcoreMesh(**_MESH),
        compiler_params=pltpu.CompilerParams(use_tc_tiling_on_sc=False),
        scratch_types=[
            pltpu.VMEM((nidx,), jnp.int32),
            pltpu.VMEM((2, L, D), jnp.float32),
            pltpu.VMEM((rows_w, D), jnp.float32),
            pltpu.SemaphoreType.DMA,
            pltpu.SemaphoreType.DMA,
        ],
    )
    def sc_pool(lin_hbm, idx_hbm, out_hbm, idx_v, buf, pooled_v, sem0, sem1):
        wid = lax.axis_index("s") * NC + lax.axis_index("c")
        pltpu.sync_copy(idx_hbm.at[pl.ds(wid * nidx, nidx)], idx_v)
        sems = (sem0, sem1)

        def pbody(i, carry):
            x = idx_v[pl.ds(i * LANES, LANES)]
            p = ((x & (-_BLK)) | ((x & (half - 1)) << 1)
                 | (lax.shift_right_logical(x, hshift) & 1))
            idx_v[pl.ds(i * LANES, LANES)] = jnp.where(x < Vt, p, x)
            return carry
        lax.fori_loop(0, nidx // LANES, pbody, 0, unroll=8)

        def row_copies(r, b):
            o = r * L
            ca = pltpu.make_async_copy(
                lin_hbm.at[idx_v.at[pl.ds(o, CA)]],
                buf.at[b, pl.ds(0, CA)], sems[b])
            cb = pltpu.make_async_copy(
                lin_hbm.at[idx_v.at[pl.ds(o + CA, CB)]],
                buf.at[b, pl.ds(CA, CB)], sems[b])
            return ca, cb

        def issue(r, b):
            ca, cb = row_copies(r, b)
            ca.start()
            cb.start()

        def wait_row(r, b):
            ca, cb = row_copies(r, b)
            ca.wait()
            cb.wait()

        def acc_row(r, b):
            def jbody(j, accs):
                return tuple(
                    accs[k] + buf[b, j, pl.ds(k * LANES, LANES)]
                    for k in range(nvec))
            z = jnp.zeros((LANES,), jnp.float32)
            accs = lax.fori_loop(0, L, jbody, (z,) * nvec, unroll=8)
            for k in range(nvec):
                pooled_v[r, pl.ds(k * LANES, LANES)] = accs[k] * scale

        issue(0, 0)
        issue(1, 1)

        def obody(rr, carry):
            for b in range(2):
                r = 2 * rr + b
                wait_row(r, b)

                @pl.when(r + 2 < rows_w)
                def _():
                    issue(r + 2, b)

                acc_row(r, b)
            return carry

        lax.fori_loop(0, rows_w // 2, obody, 0)
        pltpu.sync_copy(pooled_v, out_hbm.at[pl.ds(wid * rows_w, rows_w)])

    return sc_pool


def _dense_body(pooled_ref, w1_ref, b1_ref, w2_ref, b2_ref, out_ref):
    p = pooled_ref[...]
    h = jnp.dot(p, w1_ref[...], preferred_element_type=jnp.float32)
    h = jnp.maximum(h + b1_ref[...], 0.0)
    logits = jnp.dot(h, w2_ref[...], preferred_element_type=jnp.float32)
    logits = logits + b2_ref[...]
    m = jnp.max(logits, axis=-1, keepdims=True)
    e = jnp.exp(logits - m)
    out_ref[...] = e / jnp.sum(e, axis=-1, keepdims=True)


def kernel(inputs, emb_table, W1, b1, W2, b2):
    B, L = inputs.shape
    V, D = emb_table.shape
    H = W1.shape[1]
    C = W2.shape[1]
    CP = 128

    idx_flat = inputs.astype(jnp.int32).reshape(-1)
    vrem = V - (V // _BLK) * _BLK
    tail = emb_table[V - vrem:].reshape(vrem // 2, 2 * D)
    pair = _make_tc_format(V, D)(emb_table.T, tail)
    lin = pair.reshape(V, D)
    pooled = _make_sc_pool(B, L, D, (V // _BLK) * _BLK)(lin, idx_flat)

    w2p = jnp.zeros((H, CP), jnp.float32).at[:, :C].set(W2)
    b2p = jnp.full((1, CP), -1e30, jnp.float32).at[0, :C].set(b2)
    b1r = b1.reshape(1, H)

    out = pl.pallas_call(
        _dense_body,
        out_shape=jax.ShapeDtypeStruct((B, CP), jnp.float32),
    )(pooled, W1, b1r, w2p, b2p)
    return out[:, :C]


# speedup vs baseline: 19.8856x; 1.2714x over previous
import functools

import jax
import jax.numpy as jnp
from jax import lax
from jax.experimental import pallas as pl
from jax.experimental.pallas import tpu as pltpu
from jax.experimental.pallas import tpu_sc as plsc

NC = 2
NS = 16
NW = NC * NS
LANES = 16

_MESH = dict(core_axis_name="c", subcore_axis_name="s",
             num_cores=NC, num_subcores=NS)


_BLK = 4096


def _make_tc_format(V, D):
    nblk = V // _BLK
    vrem = V - nblk * _BLK

    def body(tbl_ref, tail_ref, out_ref):
        pid = pl.program_id(0)

        @pl.when(pid < nblk)
        def _():
            t = tbl_ref[...].T
            out_ref[:, pl.ds(0, D)] = t[: _BLK // 2]
            out_ref[:, pl.ds(D, D)] = t[_BLK // 2:]

        if vrem:
            @pl.when(pid == nblk)
            def _():
                out_ref[pl.ds(0, vrem // 2), :] = tail_ref[...]

    grid = nblk + (1 if vrem else 0)
    return pl.pallas_call(
        body,
        grid=(grid,),
        in_specs=[
            pl.BlockSpec((D, _BLK), lambda c: (0, jnp.minimum(c, nblk - 1))),
            pl.BlockSpec((vrem // 2, 2 * D) if vrem else None,
                         (lambda c: (0, 0)) if vrem else None),
        ],
        out_specs=pl.BlockSpec((_BLK // 2, 2 * D), lambda c: (c, 0)),
        out_shape=jax.ShapeDtypeStruct((V // 2, 2 * D), jnp.float32),
    )


def _make_sc_pool(B, L, D, Vt):
    rows_w = B // NW
    CA = 104
    CB = L - CA
    nvec = D // LANES
    scale = 1.0 / L
    nidx = rows_w * L
    half = _BLK // 2
    hshift = half.bit_length() - 1

    @functools.partial(
        pl.kernel,
        out_type=jax.ShapeDtypeStruct((B, D), jnp.float32),
        mesh=plsc.VectorSubcoreMesh(**_MESH),
        compiler_params=pltpu.CompilerParams(use_tc_tiling_on_sc=False),
        scratch_types=[
            pltpu.VMEM((nidx,), jnp.int32),
            pltpu.VMEM((2, L, D), jnp.float32),
            pltpu.VMEM((rows_w, D), jnp.float32),
            pltpu.SemaphoreType.DMA,
            pltpu.SemaphoreType.DMA,
        ],
    )
    def sc_pool(lin_hbm, idx_hbm, out_hbm, idx_v, buf, pooled_v, sem0, sem1):
        wid = lax.axis_index("s") * NC + lax.axis_index("c")
        pltpu.sync_copy(idx_hbm.at[pl.ds(wid * nidx, nidx)], idx_v)
        sems = (sem0, sem1)

        def pbody(i, carry):
            x = idx_v[pl.ds(i * LANES, LANES)]
            p = ((x & (-_BLK)) | ((x & (half - 1)) << 1)
                 | (lax.shift_right_logical(x, hshift) & 1))
            idx_v[pl.ds(i * LANES, LANES)] = jnp.where(x < Vt, p, x)
            return carry
        lax.fori_loop(0, nidx // LANES, pbody, 0, unroll=8)

        def row_copies(r, b):
            o = r * L
            ca = pltpu.make_async_copy(
                lin_hbm.at[idx_v.at[pl.ds(o, CA)]],
                buf.at[b, pl.ds(0, CA)], sems[b])
            cb = pltpu.make_async_copy(
                lin_hbm.at[idx_v.at[pl.ds(o + CA, CB)]],
                buf.at[b, pl.ds(CA, CB)], sems[b])
            return ca, cb

        def issue(r, b):
            ca, cb = row_copies(r, b)
            ca.start()
            cb.start()

        def wait_row(r, b):
            ca, cb = row_copies(r, b)
            ca.wait()
            cb.wait()

        def acc_row(r, b):
            def jbody(j, accs):
                return tuple(
                    accs[k] + buf[b, j, pl.ds(k * LANES, LANES)]
                    for k in range(nvec))
            z = jnp.zeros((LANES,), jnp.float32)
            accs = lax.fori_loop(0, L, jbody, (z,) * nvec, unroll=8)
            for k in range(nvec):
                pooled_v[r, pl.ds(k * LANES, LANES)] = accs[k] * scale

        issue(0, 0)
        issue(1, 1)

        def obody(rr, carry):
            for b in range(2):
                r = 2 * rr + b
                wait_row(r, b)

                @pl.when(r + 2 < rows_w)
                def _():
                    issue(r + 2, b)

                acc_row(r, b)
            return carry

        lax.fori_loop(0, rows_w // 2, obody, 0)
        pltpu.sync_copy(pooled_v, out_hbm.at[pl.ds(wid * rows_w, rows_w)])

    return sc_pool


def _dense_body(pooled_ref, w1_ref, b1_ref, w2_ref, b2_ref, out_ref):
    p = pooled_ref[...]
    h = jnp.dot(p, w1_ref[...], preferred_element_type=jnp.float32)
    h = jnp.maximum(h + b1_ref[...], 0.0)
    logits = jnp.dot(h, w2_ref[...], preferred_element_type=jnp.float32)
    logits = logits + b2_ref[...]
    m = jnp.max(logits, axis=-1, keepdims=True)
    e = jnp.exp(logits - m)
    out_ref[...] = e / jnp.sum(e, axis=-1, keepdims=True)


def kernel(inputs, emb_table, W1, b1, W2, b2):
    B, L = inputs.shape
    V, D = emb_table.shape
    H = W1.shape[1]
    C = W2.shape[1]
    CP = 128

    idx_flat = inputs.astype(jnp.int32).reshape(-1)
    vrem = V - (V // _BLK) * _BLK
    tail = emb_table[V - vrem:].reshape(vrem // 2, 2 * D)
    pair = _make_tc_format(V, D)(emb_table.T, tail)
    lin = pair.reshape(V, D)
    pooled = _make_sc_pool(B, L, D, (V // _BLK) * _BLK)(lin, idx_flat)

    w2p = jnp.zeros((H, CP), jnp.float32).at[:, :C].set(W2)
    b2p = jnp.full((1, CP), -1e30, jnp.float32).at[0, :C].set(b2)
    b1r = b1.reshape(1, H)

    out = pl.pallas_call(
        _dense_body,
        out_shape=jax.ShapeDtypeStruct((B, CP), jnp.float32),
    )(pooled, W1, b1r, w2p, b2p)
    return out[:, :C]


# speedup vs baseline: 23.4342x; 1.1784x over previous
import functools

import jax
import jax.numpy as jnp
from jax import lax
from jax.experimental import pallas as pl
from jax.experimental.pallas import tpu as pltpu
from jax.experimental.pallas import tpu_sc as plsc

NC = 2
NS = 16
NW = NC * NS
LANES = 16

_MESH = dict(core_axis_name="c", subcore_axis_name="s",
             num_cores=NC, num_subcores=NS)


_BLK = 8192


def _make_tc_format(V, D):
    nblk = V // _BLK
    vrem = V - nblk * _BLK

    def body(tbl_ref, tail_ref, out_ref):
        pid = pl.program_id(0)

        @pl.when(pid < nblk)
        def _():
            t = tbl_ref[...].T
            out_ref[:, pl.ds(0, D)] = t[: _BLK // 2]
            out_ref[:, pl.ds(D, D)] = t[_BLK // 2:]

        if vrem:
            @pl.when(pid == nblk)
            def _():
                out_ref[pl.ds(0, vrem // 2), :] = tail_ref[...]

    grid = nblk + (1 if vrem else 0)
    return pl.pallas_call(
        body,
        grid=(grid,),
        in_specs=[
            pl.BlockSpec((D, _BLK), lambda c: (0, jnp.minimum(c, nblk - 1))),
            pl.BlockSpec((vrem // 2, 2 * D) if vrem else None,
                         (lambda c: (0, 0)) if vrem else None),
        ],
        out_specs=pl.BlockSpec((_BLK // 2, 2 * D), lambda c: (c, 0)),
        out_shape=jax.ShapeDtypeStruct((V // 2, 2 * D), jnp.float32),
    )


def _make_sc_pool(B, L, D, Vt):
    rows_w = B // NW
    CA = 104
    CB = L - CA
    nvec = D // LANES
    scale = 1.0 / L
    nidx = rows_w * L
    half = _BLK // 2
    hshift = half.bit_length() - 1

    @functools.partial(
        pl.kernel,
        out_type=jax.ShapeDtypeStruct((B, D), jnp.float32),
        mesh=plsc.VectorSubcoreMesh(**_MESH),
        compiler_params=pltpu.CompilerParams(use_tc_tiling_on_sc=False),
        scratch_types=[
            pltpu.VMEM((nidx,), jnp.int32),
            pltpu.VMEM((2, L, D), jnp.float32),
            pltpu.VMEM((rows_w, D), jnp.float32),
            pltpu.SemaphoreType.DMA,
            pltpu.SemaphoreType.DMA,
        ],
    )
    def sc_pool(lin_hbm, idx_hbm, out_hbm, idx_v, buf, pooled_v, sem0, sem1):
        wid = lax.axis_index("s") * NC + lax.axis_index("c")
        pltpu.sync_copy(idx_hbm.at[pl.ds(wid * nidx, nidx)], idx_v)
        sems = (sem0, sem1)

        def pbody(i, carry):
            x = idx_v[pl.ds(i * LANES, LANES)]
            p = ((x & (-_BLK)) | ((x & (half - 1)) << 1)
                 | (lax.shift_right_logical(x, hshift) & 1))
            idx_v[pl.ds(i * LANES, LANES)] = jnp.where(x < Vt, p, x)
            return carry
        lax.fori_loop(0, nidx // LANES, pbody, 0, unroll=8)

        def row_copies(r, b):
            o = r * L
            ca = pltpu.make_async_copy(
                lin_hbm.at[idx_v.at[pl.ds(o, CA)]],
                buf.at[b, pl.ds(0, CA)], sems[b])
            cb = pltpu.make_async_copy(
                lin_hbm.at[idx_v.at[pl.ds(o + CA, CB)]],
                buf.at[b, pl.ds(CA, CB)], sems[b])
            return ca, cb

        def issue(r, b):
            ca, cb = row_copies(r, b)
            ca.start()
            cb.start()

        def wait_row(r, b):
            ca, cb = row_copies(r, b)
            ca.wait()
            cb.wait()

        def acc_row(r, b):
            def jbody(j, accs):
                return tuple(
                    accs[k] + buf[b, j, pl.ds(k * LANES, LANES)]
                    for k in range(nvec))
            z = jnp.zeros((LANES,), jnp.float32)
            accs = lax.fori_loop(0, L, jbody, (z,) * nvec, unroll=8)
            for k in range(nvec):
                pooled_v[r, pl.ds(k * LANES, LANES)] = accs[k] * scale

        issue(0, 0)
        issue(1, 1)

        def obody(rr, carry):
            for b in range(2):
                r = 2 * rr + b
                wait_row(r, b)

                @pl.when(r + 2 < rows_w)
                def _():
                    issue(r + 2, b)

                acc_row(r, b)
            return carry

        lax.fori_loop(0, rows_w // 2, obody, 0)
        pltpu.sync_copy(pooled_v, out_hbm.at[pl.ds(wid * rows_w, rows_w)])

    return sc_pool


def _dense_body(pooled_ref, w1_ref, b1_ref, w2_ref, b2_ref, out_ref):
    p = pooled_ref[...]
    h = jnp.dot(p, w1_ref[...], preferred_element_type=jnp.float32)
    h = jnp.maximum(h + b1_ref[...], 0.0)
    logits = jnp.dot(h, w2_ref[...], preferred_element_type=jnp.float32)
    logits = logits + b2_ref[...]
    m = jnp.max(logits, axis=-1, keepdims=True)
    e = jnp.exp(logits - m)
    out_ref[...] = e / jnp.sum(e, axis=-1, keepdims=True)


def kernel(inputs, emb_table, W1, b1, W2, b2):
    B, L = inputs.shape
    V, D = emb_table.shape
    H = W1.shape[1]
    C = W2.shape[1]
    CP = 128

    idx_flat = inputs.astype(jnp.int32).reshape(-1)
    vrem = V - (V // _BLK) * _BLK
    tail = emb_table[V - vrem:].reshape(vrem // 2, 2 * D)
    pair = _make_tc_format(V, D)(emb_table.T, tail)
    lin = pair.reshape(V, D)
    pooled = _make_sc_pool(B, L, D, (V // _BLK) * _BLK)(lin, idx_flat)

    w2p = jnp.zeros((H, CP), jnp.float32).at[:, :C].set(W2)
    b2p = jnp.full((1, CP), -1e30, jnp.float32).at[0, :C].set(b2)
    b1r = b1.reshape(1, H)

    out = pl.pallas_call(
        _dense_body,
        out_shape=jax.ShapeDtypeStruct((B, CP), jnp.float32),
    )(pooled, W1, b1r, w2p, b2p)
    return out[:, :C]


# speedup vs baseline: 25.7454x; 1.0986x over previous
import functools

import jax
import jax.numpy as jnp
from jax import lax
from jax.experimental import pallas as pl
from jax.experimental.pallas import tpu as pltpu
from jax.experimental.pallas import tpu_sc as plsc

NC = 2
NS = 16
NW = NC * NS
LANES = 16

_MESH = dict(core_axis_name="c", subcore_axis_name="s",
             num_cores=NC, num_subcores=NS)


_BLK = 16384


def _make_tc_format(V, D):
    nblk = V // _BLK
    vrem = V - nblk * _BLK

    def body(tbl_ref, tail_ref, out_ref):
        pid = pl.program_id(0)

        @pl.when(pid < nblk)
        def _():
            t = tbl_ref[...].T
            out_ref[:, pl.ds(0, D)] = t[: _BLK // 2]
            out_ref[:, pl.ds(D, D)] = t[_BLK // 2:]

        if vrem:
            @pl.when(pid == nblk)
            def _():
                out_ref[pl.ds(0, vrem // 2), :] = tail_ref[...]

    grid = nblk + (1 if vrem else 0)
    return pl.pallas_call(
        body,
        grid=(grid,),
        in_specs=[
            pl.BlockSpec((D, _BLK), lambda c: (0, jnp.minimum(c, nblk - 1))),
            pl.BlockSpec((vrem // 2, 2 * D) if vrem else None,
                         (lambda c: (0, 0)) if vrem else None),
        ],
        out_specs=pl.BlockSpec((_BLK // 2, 2 * D), lambda c: (c, 0)),
        out_shape=jax.ShapeDtypeStruct((V // 2, 2 * D), jnp.float32),
    )


def _make_sc_pool(B, L, D, Vt):
    rows_w = B // NW
    CA = 104
    CB = L - CA
    nvec = D // LANES
    scale = 1.0 / L
    nidx = rows_w * L
    half = _BLK // 2
    hshift = half.bit_length() - 1

    @functools.partial(
        pl.kernel,
        out_type=jax.ShapeDtypeStruct((B, D), jnp.float32),
        mesh=plsc.VectorSubcoreMesh(**_MESH),
        compiler_params=pltpu.CompilerParams(use_tc_tiling_on_sc=False),
        scratch_types=[
            pltpu.VMEM((nidx,), jnp.int32),
            pltpu.VMEM((2, L, D), jnp.float32),
            pltpu.VMEM((rows_w, D), jnp.float32),
            pltpu.SemaphoreType.DMA,
            pltpu.SemaphoreType.DMA,
        ],
    )
    def sc_pool(lin_hbm, idx_hbm, out_hbm, idx_v, buf, pooled_v, sem0, sem1):
        wid = lax.axis_index("s") * NC + lax.axis_index("c")
        pltpu.sync_copy(idx_hbm.at[pl.ds(wid * nidx, nidx)], idx_v)
        sems = (sem0, sem1)

        def pbody(i, carry):
            x = idx_v[pl.ds(i * LANES, LANES)]
            p = ((x & (-_BLK)) | ((x & (half - 1)) << 1)
                 | (lax.shift_right_logical(x, hshift) & 1))
            idx_v[pl.ds(i * LANES, LANES)] = jnp.where(x < Vt, p, x)
            return carry
        lax.fori_loop(0, nidx // LANES, pbody, 0, unroll=8)

        def row_copies(r, b):
            o = r * L
            ca = pltpu.make_async_copy(
                lin_hbm.at[idx_v.at[pl.ds(o, CA)]],
                buf.at[b, pl.ds(0, CA)], sems[b])
            cb = pltpu.make_async_copy(
                lin_hbm.at[idx_v.at[pl.ds(o + CA, CB)]],
                buf.at[b, pl.ds(CA, CB)], sems[b])
            return ca, cb

        def issue(r, b):
            ca, cb = row_copies(r, b)
            ca.start()
            cb.start()

        def wait_row(r, b):
            ca, cb = row_copies(r, b)
            ca.wait()
            cb.wait()

        def acc_row(r, b):
            def jbody(j, accs):
                return tuple(
                    accs[k] + buf[b, j, pl.ds(k * LANES, LANES)]
                    for k in range(nvec))
            z = jnp.zeros((LANES,), jnp.float32)
            accs = lax.fori_loop(0, L, jbody, (z,) * nvec, unroll=8)
            for k in range(nvec):
                pooled_v[r, pl.ds(k * LANES, LANES)] = accs[k] * scale

        issue(0, 0)
        issue(1, 1)

        def obody(rr, carry):
            for b in range(2):
                r = 2 * rr + b
                wait_row(r, b)

                @pl.when(r + 2 < rows_w)
                def _():
                    issue(r + 2, b)

                acc_row(r, b)
            return carry

        lax.fori_loop(0, rows_w // 2, obody, 0)
        pltpu.sync_copy(pooled_v, out_hbm.at[pl.ds(wid * rows_w, rows_w)])

    return sc_pool


def _dense_body(pooled_ref, w1_ref, b1_ref, w2_ref, b2_ref, out_ref):
    p = pooled_ref[...]
    h = jnp.dot(p, w1_ref[...], preferred_element_type=jnp.float32)
    h = jnp.maximum(h + b1_ref[...], 0.0)
    logits = jnp.dot(h, w2_ref[...], preferred_element_type=jnp.float32)
    logits = logits + b2_ref[...]
    m = jnp.max(logits, axis=-1, keepdims=True)
    e = jnp.exp(logits - m)
    out_ref[...] = e / jnp.sum(e, axis=-1, keepdims=True)


def kernel(inputs, emb_table, W1, b1, W2, b2):
    B, L = inputs.shape
    V, D = emb_table.shape
    H = W1.shape[1]
    C = W2.shape[1]
    CP = 128

    idx_flat = inputs.astype(jnp.int32).reshape(-1)
    vrem = V - (V // _BLK) * _BLK
    tail = emb_table[V - vrem:].reshape(vrem // 2, 2 * D)
    pair = _make_tc_format(V, D)(emb_table.T, tail)
    lin = pair.reshape(V, D)
    pooled = _make_sc_pool(B, L, D, (V // _BLK) * _BLK)(lin, idx_flat)

    w2p = jnp.zeros((H, CP), jnp.float32).at[:, :C].set(W2)
    b2p = jnp.full((1, CP), -1e30, jnp.float32).at[0, :C].set(b2)
    b1r = b1.reshape(1, H)

    out = pl.pallas_call(
        _dense_body,
        out_shape=jax.ShapeDtypeStruct((B, CP), jnp.float32),
    )(pooled, W1, b1r, w2p, b2p)
    return out[:, :C]


# speedup vs baseline: 25.9115x; 1.0065x over previous
import functools

import jax
import jax.numpy as jnp
from jax import lax
from jax.experimental import pallas as pl
from jax.experimental.pallas import tpu as pltpu
from jax.experimental.pallas import tpu_sc as plsc

NC = 2
NS = 16
NW = NC * NS
LANES = 16

_MESH = dict(core_axis_name="c", subcore_axis_name="s",
             num_cores=NC, num_subcores=NS)


_BLK = 32768


def _make_tc_format(V, D):
    nblk = V // _BLK
    vrem = V - nblk * _BLK

    def body(tbl_ref, tail_ref, out_ref):
        pid = pl.program_id(0)

        @pl.when(pid < nblk)
        def _():
            t = tbl_ref[...].T
            out_ref[:, pl.ds(0, D)] = t[: _BLK // 2]
            out_ref[:, pl.ds(D, D)] = t[_BLK // 2:]

        if vrem:
            @pl.when(pid == nblk)
            def _():
                out_ref[pl.ds(0, vrem // 2), :] = tail_ref[...]

    grid = nblk + (1 if vrem else 0)
    return pl.pallas_call(
        body,
        grid=(grid,),
        in_specs=[
            pl.BlockSpec((D, _BLK), lambda c: (0, jnp.minimum(c, nblk - 1))),
            pl.BlockSpec((vrem // 2, 2 * D) if vrem else None,
                         (lambda c: (0, 0)) if vrem else None),
        ],
        out_specs=pl.BlockSpec((_BLK // 2, 2 * D), lambda c: (c, 0)),
        out_shape=jax.ShapeDtypeStruct((V // 2, 2 * D), jnp.float32),
    )


def _make_sc_pool(B, L, D, Vt):
    rows_w = B // NW
    CA = 104
    CB = L - CA
    nvec = D // LANES
    scale = 1.0 / L
    nidx = rows_w * L
    half = _BLK // 2
    hshift = half.bit_length() - 1

    @functools.partial(
        pl.kernel,
        out_type=jax.ShapeDtypeStruct((B, D), jnp.float32),
        mesh=plsc.VectorSubcoreMesh(**_MESH),
        compiler_params=pltpu.CompilerParams(use_tc_tiling_on_sc=False),
        scratch_types=[
            pltpu.VMEM((nidx,), jnp.int32),
            pltpu.VMEM((2, L, D), jnp.float32),
            pltpu.VMEM((rows_w, D), jnp.float32),
            pltpu.SemaphoreType.DMA,
            pltpu.SemaphoreType.DMA,
        ],
    )
    def sc_pool(lin_hbm, idx_hbm, out_hbm, idx_v, buf, pooled_v, sem0, sem1):
        wid = lax.axis_index("s") * NC + lax.axis_index("c")
        pltpu.sync_copy(idx_hbm.at[pl.ds(wid * nidx, nidx)], idx_v)
        sems = (sem0, sem1)

        def pbody(i, carry):
            x = idx_v[pl.ds(i * LANES, LANES)]
            p = ((x & (-_BLK)) | ((x & (half - 1)) << 1)
                 | (lax.shift_right_logical(x, hshift) & 1))
            idx_v[pl.ds(i * LANES, LANES)] = jnp.where(x < Vt, p, x)
            return carry
        lax.fori_loop(0, nidx // LANES, pbody, 0, unroll=8)

        def row_copies(r, b):
            o = r * L
            ca = pltpu.make_async_copy(
                lin_hbm.at[idx_v.at[pl.ds(o, CA)]],
                buf.at[b, pl.ds(0, CA)], sems[b])
            cb = pltpu.make_async_copy(
                lin_hbm.at[idx_v.at[pl.ds(o + CA, CB)]],
                buf.at[b, pl.ds(CA, CB)], sems[b])
            return ca, cb

        def issue(r, b):
            ca, cb = row_copies(r, b)
            ca.start()
            cb.start()

        def wait_row(r, b):
            ca, cb = row_copies(r, b)
            ca.wait()
            cb.wait()

        def acc_row(r, b):
            def jbody(j, accs):
                return tuple(
                    accs[k] + buf[b, j, pl.ds(k * LANES, LANES)]
                    for k in range(nvec))
            z = jnp.zeros((LANES,), jnp.float32)
            accs = lax.fori_loop(0, L, jbody, (z,) * nvec, unroll=8)
            for k in range(nvec):
                pooled_v[r, pl.ds(k * LANES, LANES)] = accs[k] * scale

        issue(0, 0)
        issue(1, 1)

        def obody(rr, carry):
            for b in range(2):
                r = 2 * rr + b
                wait_row(r, b)

                @pl.when(r + 2 < rows_w)
                def _():
                    issue(r + 2, b)

                acc_row(r, b)
            return carry

        lax.fori_loop(0, rows_w // 2, obody, 0)
        pltpu.sync_copy(pooled_v, out_hbm.at[pl.ds(wid * rows_w, rows_w)])

    return sc_pool


def _dense_body(pooled_ref, w1_ref, b1_ref, w2_ref, b2_ref, out_ref):
    p = pooled_ref[...]
    h = jnp.dot(p, w1_ref[...], preferred_element_type=jnp.float32)
    h = jnp.maximum(h + b1_ref[...], 0.0)
    logits = jnp.dot(h, w2_ref[...], preferred_element_type=jnp.float32)
    logits = logits + b2_ref[...]
    m = jnp.max(logits, axis=-1, keepdims=True)
    e = jnp.exp(logits - m)
    out_ref[...] = e / jnp.sum(e, axis=-1, keepdims=True)


def kernel(inputs, emb_table, W1, b1, W2, b2):
    B, L = inputs.shape
    V, D = emb_table.shape
    H = W1.shape[1]
    C = W2.shape[1]
    CP = 128

    idx_flat = inputs.astype(jnp.int32).reshape(-1)
    vrem = V - (V // _BLK) * _BLK
    tail = emb_table[V - vrem:].reshape(vrem // 2, 2 * D)
    pair = _make_tc_format(V, D)(emb_table.T, tail)
    lin = pair.reshape(V, D)
    pooled = _make_sc_pool(B, L, D, (V // _BLK) * _BLK)(lin, idx_flat)

    w2p = jnp.zeros((H, CP), jnp.float32).at[:, :C].set(W2)
    b2p = jnp.full((1, CP), -1e30, jnp.float32).at[0, :C].set(b2)
    b1r = b1.reshape(1, H)

    out = pl.pallas_call(
        _dense_body,
        out_shape=jax.ShapeDtypeStruct((B, CP), jnp.float32),
    )(pooled, W1, b1r, w2p, b2p)
    return out[:, :C]
